# Initial kernel scaffold; baseline (speedup 1.0000x reference)
import jax
import jax.numpy as jnp
from jax.experimental import pallas as pl


def kernel(x, edge_index, batch, W1, b1, W2, b2, Wm, bm):
    raise NotImplementedError("write your pallas kernel here")


# speedup vs baseline: 12.0780x; 12.0780x over previous
import functools

import jax
import jax.numpy as jnp
from jax import lax
from jax.experimental import pallas as pl
from jax.experimental.pallas import tpu as pltpu
from jax.experimental.pallas import tpu_sc as plsc

NC = 2
NS = 16
NW = NC * NS
CHUNK = 80


def _mesh():
    return plsc.VectorSubcoreMesh(core_axis_name="c", subcore_axis_name="s")


def _make_deg_kernel(E, N):
    iters = E // (NW * CHUNK)
    rows_per_tile = N // NS

    @functools.partial(
        pl.kernel,
        mesh=_mesh(),
        out_type=jax.ShapeDtypeStruct((NC, NS, rows_per_tile, 16),
                                      jnp.float32),
        scratch_types=[
            pltpu.VMEM((CHUNK,), jnp.int32),
            pltpu.VMEM((CHUNK, 16), jnp.float32),
            pltpu.VMEM_SHARED((N, 16), jnp.float32),
        ],
    )
    def deg_kernel(col_hbm, ones_hbm, zeros_hbm, pdeg_hbm, colv, onesv, acc):
        c = lax.axis_index("c")
        s = lax.axis_index("s")
        w = c * NS + s
        r0 = s * rows_per_tile
        pltpu.sync_copy(zeros_hbm.at[s], acc.at[pl.ds(r0, rows_per_tile)])
        pltpu.sync_copy(ones_hbm, onesv)
        plsc.subcore_barrier()
        base = w * (E // NW)

        def body(i, carry):
            off = base + i * CHUNK
            pltpu.sync_copy(col_hbm.at[pl.ds(off, CHUNK)], colv)
            pltpu.sync_copy(onesv, acc.at[colv], add=True)
            return carry

        lax.fori_loop(0, iters, body, 0)
        plsc.subcore_barrier()
        pltpu.sync_copy(acc.at[pl.ds(r0, rows_per_tile)], pdeg_hbm.at[c, s])

    return deg_kernel


def _make_agg_kernel(E, N, D):
    iters = E // (NW * CHUNK)
    rows_per_tile = N // NS

    @functools.partial(
        pl.kernel,
        mesh=_mesh(),
        out_type=jax.ShapeDtypeStruct((NC, NS, rows_per_tile, D),
                                      jnp.float32),
        scratch_types=[
            pltpu.VMEM((CHUNK,), jnp.int32),
            pltpu.VMEM((CHUNK,), jnp.int32),
            pltpu.VMEM((CHUNK, D), jnp.float32),
            pltpu.VMEM_SHARED((N, D), jnp.float32),
            pltpu.SemaphoreType.DMA,
        ],
    )
    def agg_kernel(xs_hbm, row_hbm, col_hbm, zeros_hbm, part_hbm,
                   rowv, colv, rows_v, acc, sem):
        c = lax.axis_index("c")
        s = lax.axis_index("s")
        w = c * NS + s
        r0 = s * rows_per_tile
        pltpu.sync_copy(zeros_hbm.at[s], acc.at[pl.ds(r0, rows_per_tile)])
        plsc.subcore_barrier()
        base = w * (E // NW)

        def body(i, carry):
            off = base + i * CHUNK
            pltpu.sync_copy(row_hbm.at[pl.ds(off, CHUNK)], rowv)
            pltpu.sync_copy(col_hbm.at[pl.ds(off, CHUNK)], colv)
            pltpu.async_copy(xs_hbm.at[rowv], rows_v, sem).wait()
            pltpu.sync_copy(rows_v, acc.at[colv], add=True)
            return carry

        lax.fori_loop(0, iters, body, 0)
        plsc.subcore_barrier()
        pltpu.sync_copy(acc.at[pl.ds(r0, rows_per_tile)], part_hbm.at[c, s])

    return agg_kernel


def _dis_block(pd0, pd1):
    deg = pd0[:, :1] + pd1[:, :1] + 1.0
    return lax.rsqrt(deg)


def _tc_xs1_body(x_ref, w_ref, pd0_ref, pd1_ref, xs_ref):
    dis = _dis_block(pd0_ref[...], pd1_ref[...])
    xl = jnp.dot(x_ref[...], w_ref[...], preferred_element_type=jnp.float32)
    xs_ref[...] = dis * xl


def _tc_mid_body(p0_ref, p1_ref, xs_ref, pd0_ref, pd1_ref, w_ref, b_ref,
                 out_ref):
    dis = _dis_block(pd0_ref[...], pd1_ref[...])
    h = dis * (p0_ref[...] + p1_ref[...] + xs_ref[...]) + b_ref[...]
    h = jnp.maximum(h, 0.0)
    out_ref[...] = dis * jnp.dot(h, w_ref[...],
                                 preferred_element_type=jnp.float32)


def _tc_final_body(p0_ref, p1_ref, xs_ref, pd0_ref, pd1_ref, b_ref,
                   batch_ref, wm_ref, bm_ref, out_ref, ysum, cnt, *, steps, g):
    i = pl.program_id(0)
    dis = _dis_block(pd0_ref[...], pd1_ref[...])
    h = dis * (p0_ref[...] + p1_ref[...] + xs_ref[...]) + b_ref[...]
    nb = h.shape[0]
    gids = lax.broadcasted_iota(jnp.int32, (nb, g), 1)
    onehot = (batch_ref[...] == gids).astype(jnp.float32)
    ones = jnp.ones((nb, h.shape[1]), jnp.float32)
    dn = (((0,), (0,)), ((), ()))
    ys = lax.dot_general(onehot, h, dn, preferred_element_type=jnp.float32)
    cs = lax.dot_general(onehot, ones, dn, preferred_element_type=jnp.float32)

    @pl.when(i == 0)
    def _():
        ysum[...] = jnp.zeros_like(ysum)
        cnt[...] = jnp.zeros_like(cnt)

    ysum[...] += ys
    cnt[...] += cs

    @pl.when(i == steps - 1)
    def _():
        y = ysum[...] / cnt[...]
        out_ref[...] = jnp.dot(y, wm_ref[...],
                               preferred_element_type=jnp.float32) + bm_ref[...]


def kernel(x, edge_index, batch, W1, b1, W2, b2, Wm, bm):
    N, D = x.shape
    E = edge_index.shape[1]
    H = Wm.shape[1]
    G = 64
    BN = 1000
    steps = N // BN

    row = edge_index[0]
    col = edge_index[1]
    rpt = N // NS
    zeros_d = jnp.zeros((NS, rpt, D), jnp.float32)
    zeros_16 = jnp.zeros((NS, rpt, 16), jnp.float32)
    ones_16 = jnp.ones((CHUNK, 16), jnp.float32)
    b1r = b1.reshape(1, D)
    b2r = b2.reshape(1, D)
    bmr = bm.reshape(1, H)
    batch2d = batch.reshape(N, 1)

    pdeg = _make_deg_kernel(E, N)(col, ones_16, zeros_16).reshape(NC, N, 16)
    pd0 = pdeg[0]
    pd1 = pdeg[1]

    rowspec = pl.BlockSpec((BN, D), lambda i: (i, 0))
    pdspec = pl.BlockSpec((BN, 16), lambda i: (i, 0))
    wspec = pl.BlockSpec((D, D), lambda i: (0, 0))
    bspec = pl.BlockSpec((1, D), lambda i: (0, 0))

    xs1 = pl.pallas_call(
        _tc_xs1_body,
        grid=(steps,),
        in_specs=[rowspec, wspec, pdspec, pdspec],
        out_specs=rowspec,
        out_shape=jax.ShapeDtypeStruct((N, D), jnp.float32),
    )(x, W1, pd0, pd1)

    agg = _make_agg_kernel(E, N, D)

    part1 = agg(xs1, row, col, zeros_d).reshape(NC, N, D)
    xs2 = pl.pallas_call(
        _tc_mid_body,
        grid=(steps,),
        in_specs=[rowspec, rowspec, rowspec, pdspec, pdspec, wspec, bspec],
        out_specs=rowspec,
        out_shape=jax.ShapeDtypeStruct((N, D), jnp.float32),
    )(part1[0], part1[1], xs1, pd0, pd1, W2, b1r)

    part2 = agg(xs2, row, col, zeros_d).reshape(NC, N, D)
    out = pl.pallas_call(
        functools.partial(_tc_final_body, steps=steps, g=G),
        grid=(steps,),
        in_specs=[rowspec, rowspec, rowspec, pdspec, pdspec, bspec,
                  pl.BlockSpec((BN, 1), lambda i: (i, 0)),
                  pl.BlockSpec((D, H), lambda i: (0, 0)),
                  pl.BlockSpec((1, H), lambda i: (0, 0))],
        out_specs=pl.BlockSpec((G, H), lambda i: (0, 0)),
        out_shape=jax.ShapeDtypeStruct((G, H), jnp.float32),
        scratch_shapes=[pltpu.VMEM((G, D), jnp.float32),
                        pltpu.VMEM((G, D), jnp.float32)],
    )(part2[0], part2[1], xs2, pd0, pd1, b2r, batch2d, Wm, bmr)

    return out


# speedup vs baseline: 13.3871x; 1.1084x over previous
import functools

import jax
import jax.numpy as jnp
from jax import lax
from jax.experimental import pallas as pl
from jax.experimental.pallas import tpu as pltpu
from jax.experimental.pallas import tpu_sc as plsc

NC = 2
NS = 16
NW = NC * NS
CHUNK = 80


def _mesh():
    return plsc.VectorSubcoreMesh(core_axis_name="c", subcore_axis_name="s")


def _make_agg_kernel(E, N, D):
    iters = E // (NW * CHUNK)
    rows_per_tile = N // NS

    @functools.partial(
        pl.kernel,
        mesh=_mesh(),
        out_type=jax.ShapeDtypeStruct((NC, NS, rows_per_tile, D),
                                      jnp.float32),
        scratch_types=[
            pltpu.VMEM((iters, CHUNK), jnp.int32),
            pltpu.VMEM((iters, CHUNK), jnp.int32),
            pltpu.VMEM((CHUNK, D), jnp.float32),
            pltpu.VMEM_SHARED((N, D), jnp.float32),
            pltpu.SemaphoreType.DMA,
        ],
    )
    def agg_kernel(xs_hbm, row_hbm, col_hbm, zeros_hbm, part_hbm,
                   rowm, colm, rows_v, acc, sem):
        c = lax.axis_index("c")
        s = lax.axis_index("s")
        w = c * NS + s
        r0 = s * rows_per_tile
        pltpu.sync_copy(zeros_hbm.at[s], acc.at[pl.ds(r0, rows_per_tile)])
        pltpu.sync_copy(row_hbm.at[w], rowm)
        pltpu.sync_copy(col_hbm.at[w], colm)
        plsc.subcore_barrier()

        def body(i, carry):
            pltpu.async_copy(xs_hbm.at[rowm.at[i]], rows_v, sem).wait()
            pltpu.sync_copy(rows_v, acc.at[colm.at[i]], add=True)
            return carry

        lax.fori_loop(0, iters, body, 0)
        plsc.subcore_barrier()
        pltpu.sync_copy(acc.at[pl.ds(r0, rows_per_tile)], part_hbm.at[c, s])

    return agg_kernel


def _dis_block(pd0, pd1):
    deg = pd0[:, :1] + pd1[:, :1] + 1.0
    return lax.rsqrt(deg)


def _tc_xs1_body(x_ref, w_ref, pd0_ref, pd1_ref, xs_ref):
    dis = _dis_block(pd0_ref[...], pd1_ref[...])
    xl = jnp.dot(x_ref[...], w_ref[...], preferred_element_type=jnp.float32)
    xs_ref[...] = dis * xl


def _tc_mid_body(p0_ref, p1_ref, xs_ref, pd0_ref, pd1_ref, w_ref, b_ref,
                 out_ref):
    dis = _dis_block(pd0_ref[...], pd1_ref[...])
    h = dis * (p0_ref[...] + p1_ref[...] + xs_ref[...]) + b_ref[...]
    h = jnp.maximum(h, 0.0)
    out_ref[...] = dis * jnp.dot(h, w_ref[...],
                                 preferred_element_type=jnp.float32)


def _tc_final_body(p0_ref, p1_ref, xs_ref, pd0_ref, pd1_ref, b_ref,
                   batch_ref, wm_ref, bm_ref, out_ref, ysum, cnt, *, steps, g):
    i = pl.program_id(0)
    dis = _dis_block(pd0_ref[...], pd1_ref[...])
    h = dis * (p0_ref[...] + p1_ref[...] + xs_ref[...]) + b_ref[...]
    nb = h.shape[0]
    gids = lax.broadcasted_iota(jnp.int32, (nb, g), 1)
    onehot = (batch_ref[...] == gids).astype(jnp.float32)
    ones = jnp.ones((nb, h.shape[1]), jnp.float32)
    dn = (((0,), (0,)), ((), ()))
    ys = lax.dot_general(onehot, h, dn, preferred_element_type=jnp.float32)
    cs = lax.dot_general(onehot, ones, dn, preferred_element_type=jnp.float32)

    @pl.when(i == 0)
    def _():
        ysum[...] = jnp.zeros_like(ysum)
        cnt[...] = jnp.zeros_like(cnt)

    ysum[...] += ys
    cnt[...] += cs

    @pl.when(i == steps - 1)
    def _():
        y = ysum[...] / cnt[...]
        out_ref[...] = jnp.dot(y, wm_ref[...],
                               preferred_element_type=jnp.float32) + bm_ref[...]


def kernel(x, edge_index, batch, W1, b1, W2, b2, Wm, bm):
    N, D = x.shape
    E = edge_index.shape[1]
    H = Wm.shape[1]
    G = 64
    BN = 1000
    steps = N // BN

    ipw = E // (NW * CHUNK)
    row3 = edge_index[0].reshape(NW, ipw, CHUNK)
    col3 = edge_index[1].reshape(NW, ipw, CHUNK)
    rpt = N // NS
    zeros_d = jnp.zeros((NS, rpt, D), jnp.float32)
    ones_d = jnp.ones((N, D), jnp.float32)
    b1r = b1.reshape(1, D)
    b2r = b2.reshape(1, D)
    bmr = bm.reshape(1, H)
    batch2d = batch.reshape(N, 1)

    agg = _make_agg_kernel(E, N, D)

    pdeg = agg(ones_d, col3, col3, zeros_d).reshape(NC, N, D)
    pd0 = pdeg[0]
    pd1 = pdeg[1]

    rowspec = pl.BlockSpec((BN, D), lambda i: (i, 0))
    wspec = pl.BlockSpec((D, D), lambda i: (0, 0))
    bspec = pl.BlockSpec((1, D), lambda i: (0, 0))

    xs1 = pl.pallas_call(
        _tc_xs1_body,
        grid=(steps,),
        in_specs=[rowspec, wspec, rowspec, rowspec],
        out_specs=rowspec,
        out_shape=jax.ShapeDtypeStruct((N, D), jnp.float32),
    )(x, W1, pd0, pd1)

    part1 = agg(xs1, row3, col3, zeros_d).reshape(NC, N, D)
    xs2 = pl.pallas_call(
        _tc_mid_body,
        grid=(steps,),
        in_specs=[rowspec, rowspec, rowspec, rowspec, rowspec, wspec, bspec],
        out_specs=rowspec,
        out_shape=jax.ShapeDtypeStruct((N, D), jnp.float32),
    )(part1[0], part1[1], xs1, pd0, pd1, W2, b1r)

    part2 = agg(xs2, row3, col3, zeros_d).reshape(NC, N, D)
    out = pl.pallas_call(
        functools.partial(_tc_final_body, steps=steps, g=G),
        grid=(steps,),
        in_specs=[rowspec, rowspec, rowspec, rowspec, rowspec, bspec,
                  pl.BlockSpec((BN, 1), lambda i: (i, 0)),
                  pl.BlockSpec((D, H), lambda i: (0, 0)),
                  pl.BlockSpec((1, H), lambda i: (0, 0))],
        out_specs=pl.BlockSpec((G, H), lambda i: (0, 0)),
        out_shape=jax.ShapeDtypeStruct((G, H), jnp.float32),
        scratch_shapes=[pltpu.VMEM((G, D), jnp.float32),
                        pltpu.VMEM((G, D), jnp.float32)],
    )(part2[0], part2[1], xs2, pd0, pd1, b2r, batch2d, Wm, bmr)

    return out


# speedup vs baseline: 15.2478x; 1.1390x over previous
import functools

import jax
import jax.numpy as jnp
from jax import lax
from jax.experimental import pallas as pl
from jax.experimental.pallas import tpu as pltpu
from jax.experimental.pallas import tpu_sc as plsc

NC = 2
NS = 16
NW = NC * NS
CHUNK = 125


def _mesh():
    return plsc.VectorSubcoreMesh(core_axis_name="c", subcore_axis_name="s")


def _make_agg_kernel(E, N, D):
    iters = E // (NW * CHUNK)
    rows_per_tile = N // NS

    @functools.partial(
        pl.kernel,
        mesh=_mesh(),
        out_type=jax.ShapeDtypeStruct((NC, NS, rows_per_tile, D),
                                      jnp.float32),
        scratch_types=[
            pltpu.VMEM((iters, CHUNK), jnp.int32),
            pltpu.VMEM((iters, CHUNK), jnp.int32),
            pltpu.VMEM((CHUNK, D), jnp.float32),
            pltpu.VMEM_SHARED((N, D), jnp.float32),
            pltpu.SemaphoreType.DMA,
        ],
    )
    def agg_kernel(xs_hbm, row_hbm, col_hbm, zeros_hbm, part_hbm,
                   rowm, colm, rows_v, acc, sem):
        c = lax.axis_index("c")
        s = lax.axis_index("s")
        w = c * NS + s
        r0 = s * rows_per_tile
        pltpu.sync_copy(zeros_hbm.at[s], acc.at[pl.ds(r0, rows_per_tile)])
        pltpu.sync_copy(row_hbm.at[w], rowm)
        pltpu.sync_copy(col_hbm.at[w], colm)
        plsc.subcore_barrier()

        def body(i, carry):
            pltpu.async_copy(xs_hbm.at[rowm.at[i]], rows_v, sem).wait()
            pltpu.sync_copy(rows_v, acc.at[colm.at[i]], add=True)
            return carry

        lax.fori_loop(0, iters, body, 0)
        plsc.subcore_barrier()
        pltpu.sync_copy(acc.at[pl.ds(r0, rows_per_tile)], part_hbm.at[c, s])

    return agg_kernel


def _dis_block(pd0, pd1):
    deg = pd0[:, :1] + pd1[:, :1] + 1.0
    return lax.rsqrt(deg)


def _tc_mm_body(x_ref, w_ref, xl_ref):
    xl_ref[...] = jnp.dot(x_ref[...], w_ref[...],
                          preferred_element_type=jnp.float32)


def _tc_scale_body(xl_ref, pd0_ref, pd1_ref, xs_ref):
    dis = _dis_block(pd0_ref[...], pd1_ref[...])
    xs_ref[...] = dis * xl_ref[...]


def _tc_mid_body(p0_ref, p1_ref, xs_ref, pd0_ref, pd1_ref, w_ref, b_ref,
                 out_ref):
    dis = _dis_block(pd0_ref[...], pd1_ref[...])
    h = dis * (p0_ref[...] + p1_ref[...] + xs_ref[...]) + b_ref[...]
    h = jnp.maximum(h, 0.0)
    out_ref[...] = dis * jnp.dot(h, w_ref[...],
                                 preferred_element_type=jnp.float32)


def _tc_final_body(p0_ref, p1_ref, xs_ref, pd0_ref, pd1_ref, b_ref,
                   batch_ref, wm_ref, bm_ref, out_ref, ysum, cnt, *, steps, g):
    i = pl.program_id(0)
    dis = _dis_block(pd0_ref[...], pd1_ref[...])
    h = dis * (p0_ref[...] + p1_ref[...] + xs_ref[...]) + b_ref[...]
    nb = h.shape[0]
    gids = lax.broadcasted_iota(jnp.int32, (nb, g), 1)
    onehot = (batch_ref[...] == gids).astype(jnp.float32)
    ones = jnp.ones((nb, h.shape[1]), jnp.float32)
    dn = (((0,), (0,)), ((), ()))
    ys = lax.dot_general(onehot, h, dn, preferred_element_type=jnp.float32)
    cs = lax.dot_general(onehot, ones, dn, preferred_element_type=jnp.float32)

    @pl.when(i == 0)
    def _():
        ysum[...] = jnp.zeros_like(ysum)
        cnt[...] = jnp.zeros_like(cnt)

    ysum[...] += ys
    cnt[...] += cs

    @pl.when(i == steps - 1)
    def _():
        y = ysum[...] / cnt[...]
        out_ref[...] = jnp.dot(y, wm_ref[...],
                               preferred_element_type=jnp.float32) + bm_ref[...]


def kernel(x, edge_index, batch, W1, b1, W2, b2, Wm, bm):
    N, D = x.shape
    E = edge_index.shape[1]
    H = Wm.shape[1]
    G = 64
    BN = 1000
    steps = N // BN

    ipw = E // (NW * CHUNK)
    row3 = edge_index[0].reshape(NW, ipw, CHUNK)
    col3 = edge_index[1].reshape(NW, ipw, CHUNK)
    rpt = N // NS
    zeros_d = jnp.zeros((NS, rpt, D), jnp.float32)
    ones_d = jnp.ones((N, D), jnp.float32)
    b1r = b1.reshape(1, D)
    b2r = b2.reshape(1, D)
    bmr = bm.reshape(1, H)
    batch2d = batch.reshape(N, 1)

    agg = _make_agg_kernel(E, N, D)

    pdeg = agg(ones_d, col3, col3, zeros_d).reshape(NC, N, D)
    pd0 = pdeg[0]
    pd1 = pdeg[1]

    rowspec = pl.BlockSpec((BN, D), lambda i: (i, 0))
    wspec = pl.BlockSpec((D, D), lambda i: (0, 0))
    bspec = pl.BlockSpec((1, D), lambda i: (0, 0))

    xl1 = pl.pallas_call(
        _tc_mm_body,
        grid=(steps,),
        in_specs=[rowspec, wspec],
        out_specs=rowspec,
        out_shape=jax.ShapeDtypeStruct((N, D), jnp.float32),
    )(x, W1)
    xs1 = pl.pallas_call(
        _tc_scale_body,
        grid=(steps,),
        in_specs=[rowspec, rowspec, rowspec],
        out_specs=rowspec,
        out_shape=jax.ShapeDtypeStruct((N, D), jnp.float32),
    )(xl1, pd0, pd1)

    part1 = agg(xs1, row3, col3, zeros_d).reshape(NC, N, D)
    xs2 = pl.pallas_call(
        _tc_mid_body,
        grid=(steps,),
        in_specs=[rowspec, rowspec, rowspec, rowspec, rowspec, wspec, bspec],
        out_specs=rowspec,
        out_shape=jax.ShapeDtypeStruct((N, D), jnp.float32),
    )(part1[0], part1[1], xs1, pd0, pd1, W2, b1r)

    part2 = agg(xs2, row3, col3, zeros_d).reshape(NC, N, D)
    out = pl.pallas_call(
        functools.partial(_tc_final_body, steps=steps, g=G),
        grid=(steps,),
        in_specs=[rowspec, rowspec, rowspec, rowspec, rowspec, bspec,
                  pl.BlockSpec((BN, 1), lambda i: (i, 0)),
                  pl.BlockSpec((D, H), lambda i: (0, 0)),
                  pl.BlockSpec((1, H), lambda i: (0, 0))],
        out_specs=pl.BlockSpec((G, H), lambda i: (0, 0)),
        out_shape=jax.ShapeDtypeStruct((G, H), jnp.float32),
        scratch_shapes=[pltpu.VMEM((G, D), jnp.float32),
                        pltpu.VMEM((G, D), jnp.float32)],
    )(part2[0], part2[1], xs2, pd0, pd1, b2r, batch2d, Wm, bmr)

    return out
